# Initial kernel scaffold; baseline (speedup 1.0000x reference)
import jax
import jax.numpy as jnp
from jax.experimental import pallas as pl


def kernel(pe, pos):
    raise NotImplementedError("write your pallas kernel here")


# speedup vs baseline: 2.1359x; 2.1359x over previous
import functools

import jax
import jax.numpy as jnp
from jax import lax
from jax.experimental import pallas as pl
from jax.experimental.pallas import tpu as pltpu
from jax.experimental.pallas import tpu_sc as plsc

EMBEDDING_DIM = 1024
N_INDICES = 4 * 8192

_info = plsc.get_sparse_core_info()
NC, NS = _info.num_cores, _info.num_subcores
NW = NC * NS
PER_W = N_INDICES // NW
CHUNK = 64
N_CHUNKS = PER_W // CHUNK


def _sc_gather(pe, pos_flat):
    mesh = plsc.VectorSubcoreMesh(core_axis_name="c", subcore_axis_name="s")

    @functools.partial(
        pl.kernel,
        out_type=jax.ShapeDtypeStruct((N_INDICES, EMBEDDING_DIM), jnp.float32),
        mesh=mesh,
        scratch_types=[
            pltpu.VMEM((CHUNK,), jnp.int32),
            pltpu.VMEM((CHUNK, EMBEDDING_DIM), jnp.float32),
            pltpu.SemaphoreType.DMA,
        ],
    )
    def k(table_hbm, idx_hbm, out_hbm, idx_v, rows_v, sem):
        wid = lax.axis_index("s") * NC + lax.axis_index("c")
        base = wid * PER_W

        def chunk_body(c, carry):
            off = base + c * CHUNK
            pltpu.sync_copy(idx_hbm.at[pl.ds(off, CHUNK)], idx_v)
            pltpu.async_copy(table_hbm.at[idx_v], rows_v, sem).wait()
            pltpu.sync_copy(rows_v, out_hbm.at[pl.ds(off, CHUNK)])
            return carry

        lax.fori_loop(0, N_CHUNKS, chunk_body, 0)

    return k(pe, pos_flat)


def kernel(pe, pos):
    pos_flat = pos.reshape(-1).astype(jnp.int32)
    out = _sc_gather(pe, pos_flat)
    return out.reshape((*pos.shape, EMBEDDING_DIM))


# speedup vs baseline: 2.2558x; 1.0562x over previous
import functools

import jax
import jax.numpy as jnp
from jax import lax
from jax.experimental import pallas as pl
from jax.experimental.pallas import tpu as pltpu
from jax.experimental.pallas import tpu_sc as plsc

EMBEDDING_DIM = 1024
N_INDICES = 4 * 8192

_info = plsc.get_sparse_core_info()
NC, NS = _info.num_cores, _info.num_subcores
NW = NC * NS
PER_W = N_INDICES // NW
CHUNK = 32
N_CHUNKS = PER_W // CHUNK
NBUF = 2


def _sc_gather(pe, pos_flat):
    mesh = plsc.VectorSubcoreMesh(core_axis_name="c", subcore_axis_name="s")

    @functools.partial(
        pl.kernel,
        out_type=jax.ShapeDtypeStruct((N_INDICES, EMBEDDING_DIM), jnp.float32),
        mesh=mesh,
        scratch_types=[
            pltpu.VMEM((PER_W,), jnp.int32),
            pltpu.VMEM((NBUF, CHUNK, EMBEDDING_DIM), jnp.float32),
            pltpu.SemaphoreType.DMA,
            pltpu.SemaphoreType.DMA,
            pltpu.SemaphoreType.DMA,
            pltpu.SemaphoreType.DMA,
        ],
    )
    def k(table_hbm, idx_hbm, out_hbm, idx_v, rows_v, g0, g1, w0, w1):
        wid = lax.axis_index("s") * NC + lax.axis_index("c")
        base = wid * PER_W
        gsem = (g0, g1)
        wsem = (w0, w1)

        pltpu.sync_copy(idx_hbm.at[pl.ds(base, PER_W)], idx_v)

        def start_gather(c, b):
            pltpu.async_copy(
                table_hbm.at[idx_v.at[pl.ds(c * CHUNK, CHUNK)]],
                rows_v.at[b], gsem[b])

        def wait_gather(b):
            pltpu.make_async_copy(table_hbm.at[idx_v.at[pl.ds(0, CHUNK)]],
                                  rows_v.at[b], gsem[b]).wait()

        def start_write(c, b):
            pltpu.async_copy(rows_v.at[b],
                             out_hbm.at[pl.ds(base + c * CHUNK, CHUNK)], wsem[b])

        def wait_write(b):
            pltpu.make_async_copy(rows_v.at[b],
                                  out_hbm.at[pl.ds(0, CHUNK)], wsem[b]).wait()

        for b in range(NBUF):
            start_gather(b, b)

        def body(i, carry):
            for b in range(NBUF):
                wait_gather(b)
                start_write(NBUF * i + b, b)
            for b in range(NBUF):
                c = NBUF * i + b
                wait_write(b)

                @pl.when(c + NBUF < N_CHUNKS)
                def _():
                    start_gather(c + NBUF, b)

            return carry

        lax.fori_loop(0, N_CHUNKS // NBUF, body, 0)

    return k(pe, pos_flat)


def kernel(pe, pos):
    pos_flat = pos.reshape(-1).astype(jnp.int32)
    out = _sc_gather(pe, pos_flat)
    return out.reshape((*pos.shape, EMBEDDING_DIM))


# speedup vs baseline: 2.3315x; 1.0335x over previous
import functools

import jax
import jax.numpy as jnp
from jax import lax
from jax.experimental import pallas as pl
from jax.experimental.pallas import tpu as pltpu
from jax.experimental.pallas import tpu_sc as plsc

EMBEDDING_DIM = 1024
N_INDICES = 4 * 8192

_info = plsc.get_sparse_core_info()
NC, NS = _info.num_cores, _info.num_subcores
NW = NC * NS
PER_W = N_INDICES // NW
CHUNK = 16
N_CHUNKS = PER_W // CHUNK
NBUF = 4


def _sc_gather(pe, pos_flat):
    mesh = plsc.VectorSubcoreMesh(core_axis_name="c", subcore_axis_name="s")

    @functools.partial(
        pl.kernel,
        out_type=jax.ShapeDtypeStruct((N_INDICES, EMBEDDING_DIM), jnp.float32),
        mesh=mesh,
        scratch_types=[
            pltpu.VMEM((PER_W,), jnp.int32),
            pltpu.VMEM((NBUF, CHUNK, EMBEDDING_DIM), jnp.float32),
        ] + [pltpu.SemaphoreType.DMA] * (2 * NBUF),
    )
    def k(table_hbm, idx_hbm, out_hbm, idx_v, rows_v, *sems):
        wid = lax.axis_index("s") * NC + lax.axis_index("c")
        base = wid * PER_W
        gsem = sems[:NBUF]
        wsem = sems[NBUF:]

        pltpu.sync_copy(idx_hbm.at[pl.ds(base, PER_W)], idx_v)

        def start_gather(c, b):
            pltpu.async_copy(
                table_hbm.at[idx_v.at[pl.ds(c * CHUNK, CHUNK)]],
                rows_v.at[b], gsem[b])

        def wait_gather(b):
            pltpu.make_async_copy(table_hbm.at[idx_v.at[pl.ds(0, CHUNK)]],
                                  rows_v.at[b], gsem[b]).wait()

        def start_write(c, b):
            pltpu.async_copy(rows_v.at[b],
                             out_hbm.at[pl.ds(base + c * CHUNK, CHUNK)], wsem[b])

        def wait_write(b):
            pltpu.make_async_copy(rows_v.at[b],
                                  out_hbm.at[pl.ds(0, CHUNK)], wsem[b]).wait()

        for b in range(NBUF):
            start_gather(b, b)

        def body(i, carry):
            for b in range(NBUF):
                wait_gather(b)
                start_write(NBUF * i + b, b)
            for b in range(NBUF):
                c = NBUF * i + b
                wait_write(b)

                @pl.when(c + NBUF < N_CHUNKS)
                def _():
                    start_gather(c + NBUF, b)

            return carry

        lax.fori_loop(0, N_CHUNKS // NBUF, body, 0)

    return k(pe, pos_flat)


def kernel(pe, pos):
    pos_flat = pos.reshape(-1).astype(jnp.int32)
    out = _sc_gather(pe, pos_flat)
    return out.reshape((*pos.shape, EMBEDDING_DIM))
